# Initial kernel scaffold; baseline (speedup 1.0000x reference)
import jax
import jax.numpy as jnp
from jax.experimental import pallas as pl


def kernel(user_id, gender, age, occupation, zip_code, user_click_item_seq, user_click_item_seq_length, pos_item_sample, neg_item_sample, user_emb, gender_emb, age_emb, occupation_emb, zip_emb, item_emb, W1, b1):
    raise NotImplementedError("write your pallas kernel here")


# speedup vs baseline: 2.1098x; 2.1098x over previous
import jax
import jax.numpy as jnp
from jax import lax
from jax.experimental import pallas as pl
from jax.experimental.pallas import tpu as pltpu
from jax.experimental.pallas import tpu_sc as plsc

B = 200
L = 200
D = 64
N = 100
V_ITEM = 3707
V_PAD = 3712
NPN = 112
RPW = 8
NWORK = B // RPW
CH = 112
NCH = 2
SEQ_PAD = NCH * CH
DUMP = 16 * RPW

_mesh = plsc.VectorSubcoreMesh(core_axis_name="c", subcore_axis_name="s")


def _sc1_body(uid, gid, aid, oid, zid, seq, slen,
              ut, gt, at_, ot, zt, it, feats,
              idq, rows, sidx, lenv, buf, idx2, zb, pb, pool,
              gsem, sem0, sem1):
    c = lax.axis_index("c")
    s = lax.axis_index("s")
    wid = s * 2 + c

    @pl.when(wid < NWORK)
    def _():
        base = wid * RPW
        for t, (tab, ids) in enumerate(
                [(ut, uid), (gt, gid), (at_, aid), (ot, oid), (zt, zid)]):
            pltpu.sync_copy(ids.at[pl.ds(base, RPW)], idq)
            pltpu.async_copy(tab.at[idq], rows, gsem).wait()
            pltpu.sync_copy(rows, feats.at[t, pl.ds(base, RPW)])
        zeros16 = jnp.zeros((16,), jnp.float32)
        for i in range(RPW):
            for d4 in range(D // 16):
                zb[i, pl.ds(d4 * 16, 16)] = zeros16
        pltpu.sync_copy(zb, pool.at[pl.ds(s * RPW, RPW)])
        pltpu.sync_copy(seq.at[pl.ds(base, RPW)], sidx)
        pltpu.sync_copy(slen.at[pl.ds(base, RPW)], lenv.at[pl.ds(0, RPW)])
        lv = lenv[...]
        sems = [sem0, sem1]
        cps = [pltpu.async_copy(it.at[sidx.at[0]], buf.at[0, pl.ds(0, L)], sem0),
               None]
        dumpv = jnp.full((16,), DUMP, jnp.int32)
        for r in range(RPW):
            if r + 1 < RPW:
                cps[(r + 1) % 2] = pltpu.async_copy(
                    it.at[sidx.at[r + 1]],
                    buf.at[(r + 1) % 2, pl.ds(0, L)], sems[(r + 1) % 2])
            lb = jnp.broadcast_to(lv[r], (16,))
            slotv = jnp.full((16,), s * RPW + r, jnp.int32)
            for rr in range(NCH):
                for ch in range(CH // 16):
                    jv = lax.iota(jnp.int32, 16) + (rr * CH + ch * 16)
                    idx2[rr, pl.ds(ch * 16, 16)] = jnp.where(jv < lb, slotv, dumpv)
            cps[r % 2].wait()
            for rr in range(NCH):
                pltpu.sync_copy(buf.at[r % 2, pl.ds(rr * CH, CH)],
                                pool.at[idx2.at[rr]], add=True)
        pltpu.sync_copy(pool.at[pl.ds(s * RPW, RPW)], pb)
        pltpu.sync_copy(pb, feats.at[5, pl.ds(base, RPW)])


_sc1 = pl.kernel(
    _sc1_body,
    out_type=jax.ShapeDtypeStruct((6, B, D), jnp.float32),
    mesh=_mesh,
    compiler_params=pltpu.CompilerParams(use_tc_tiling_on_sc=False, needs_layout_passes=False),
    scratch_types=[
        pltpu.VMEM((RPW,), jnp.int32),
        pltpu.VMEM((RPW, D), jnp.float32),
        pltpu.VMEM((RPW, L), jnp.int32),
        pltpu.VMEM((16,), jnp.int32),
        pltpu.VMEM((2, SEQ_PAD, D), jnp.float32),
        pltpu.VMEM((NCH, CH), jnp.int32),
        pltpu.VMEM((RPW, D), jnp.float32),
        pltpu.VMEM((RPW, D), jnp.float32),
        pltpu.VMEM_SHARED((16 * RPW + 8, D), jnp.float32),
        pltpu.SemaphoreType.DMA,
        pltpu.SemaphoreType.DMA,
        pltpu.SemaphoreType.DMA,
    ],
)


def _tc_body(x_ref, w_ref, b_ref, itT_ref, out_ref):
    u = jnp.dot(x_ref[0], w_ref[0], preferred_element_type=jnp.float32,
                precision=lax.Precision.HIGHEST)
    for f in range(1, 6):
        u = u + jnp.dot(x_ref[f], w_ref[f], preferred_element_type=jnp.float32,
                        precision=lax.Precision.HIGHEST)
    u = jnp.maximum(u + b_ref[...], 0.0)
    out_ref[...] = jnp.dot(u, itT_ref[...], preferred_element_type=jnp.float32,
                           precision=lax.Precision.HIGHEST)


def _sc2_body(sflat, pn, out, pnb, pnf, srow, ob, gsem):
    c = lax.axis_index("c")
    s = lax.axis_index("s")
    wid = s * 2 + c

    @pl.when(wid < NWORK)
    def _():
        base = wid * RPW
        pltpu.sync_copy(pn.at[pl.ds(base, RPW)], pnb)
        nchunks = NPN // 16
        lanemask = (lax.iota(jnp.int32, 16) + (nchunks - 1) * 16) < (N + 1)
        for r in range(RPW):
            off = (base + r) * V_PAD
            for ch in range(nchunks):
                pnf[pl.ds(ch * 16, 16)] = pnb[r, pl.ds(ch * 16, 16)] + off
            pltpu.async_copy(sflat.at[pnf], srow, gsem).wait()
            chunks = [srow[pl.ds(ch * 16, 16)] for ch in range(nchunks)]
            neg_inf = jnp.full((16,), -3e38, jnp.float32)
            chunks[-1] = jnp.where(lanemask, chunks[-1], neg_inf)
            m = chunks[0]
            for ch in range(1, nchunks):
                m = jnp.maximum(m, chunks[ch])
            ms = jnp.max(m)
            es = [jnp.exp(cv - ms) for cv in chunks]
            es[-1] = jnp.where(lanemask, es[-1], jnp.zeros((16,), jnp.float32))
            tot = es[0]
            for ch in range(1, nchunks):
                tot = tot + es[ch]
            denom = jnp.broadcast_to(jnp.sum(tot), (16,))
            inv = jnp.ones((16,), jnp.float32) / denom
            for ch in range(nchunks):
                ob[r, pl.ds(ch * 16, 16)] = es[ch] * inv
        pltpu.sync_copy(ob, out.at[pl.ds(base, RPW)])


_sc2 = pl.kernel(
    _sc2_body,
    out_type=jax.ShapeDtypeStruct((B, NPN), jnp.float32),
    mesh=_mesh,
    compiler_params=pltpu.CompilerParams(use_tc_tiling_on_sc=False, needs_layout_passes=False),
    scratch_types=[
        pltpu.VMEM((RPW, NPN), jnp.int32),
        pltpu.VMEM((NPN,), jnp.int32),
        pltpu.VMEM((NPN,), jnp.float32),
        pltpu.VMEM((RPW, NPN), jnp.float32),
        pltpu.SemaphoreType.DMA,
    ],
)


def kernel(user_id, gender, age, occupation, zip_code, user_click_item_seq,
           user_click_item_seq_length, pos_item_sample, neg_item_sample,
           user_emb, gender_emb, age_emb, occupation_emb, zip_emb, item_emb,
           W1, b1):
    i32 = lambda x: x.astype(jnp.int32)
    feats = _sc1(i32(user_id), i32(gender), i32(age), i32(occupation),
                 i32(zip_code), i32(user_click_item_seq),
                 i32(user_click_item_seq_length),
                 user_emb, gender_emb, age_emb, occupation_emb, zip_emb,
                 item_emb)
    itT = jnp.pad(item_emb, ((0, V_PAD - V_ITEM), (0, 0))).T
    w6 = W1.reshape(6, D, D)
    scores = pl.pallas_call(
        _tc_body,
        out_shape=jax.ShapeDtypeStruct((B, V_PAD), jnp.float32),
    )(feats, w6, b1.reshape(1, D), itT)
    pn = jnp.concatenate([i32(pos_item_sample), i32(neg_item_sample)], axis=1)
    pn = jnp.pad(pn, ((0, 0), (0, NPN - (N + 1))))
    probs = _sc2(scores.reshape(B * V_PAD), pn)
    return probs[:, :N + 1].reshape(B, 1, N + 1)


# speedup vs baseline: 2.5854x; 1.2255x over previous
import jax
import jax.numpy as jnp
from jax import lax
from jax.experimental import pallas as pl
from jax.experimental.pallas import tpu as pltpu
from jax.experimental.pallas import tpu_sc as plsc

B = 200
L = 200
D = 64
N = 100
V_ITEM = 3707
V_PAD = 3712
NPN = 112
RPW = 8
NWORK = B // RPW
CH = 112
CHB = L - CH
NCH = 2
SEQ_PAD = NCH * CH
DUMP = 16 * RPW

_mesh = plsc.VectorSubcoreMesh(core_axis_name="c", subcore_axis_name="s")
_params = pltpu.CompilerParams(use_tc_tiling_on_sc=False,
                               needs_layout_passes=False)


def _sc1_body(ids5, seq, slen, ut, gt, at_, ot, zt, it, feats,
              idq5, rows5, sidx, lenv, buf, idx2, zb, pb, pool,
              sA, sG, sW, sS, *gsems):
    c = lax.axis_index("c")
    s = lax.axis_index("s")
    wid = s * 2 + c

    @pl.when(wid < NWORK)
    def _():
        base = wid * RPW
        d_ids = pltpu.async_copy(ids5.at[:, pl.ds(base, RPW)], idq5, sA)
        d_seq = pltpu.async_copy(seq.at[pl.ds(base, RPW)], sidx, sA)
        d_len = pltpu.async_copy(slen.at[pl.ds(base, RPW)],
                                 lenv.at[pl.ds(0, RPW)], sA)
        zeros16 = jnp.zeros((16,), jnp.float32)
        for i in range(RPW):
            for d4 in range(D // 16):
                zb[i, pl.ds(d4 * 16, 16)] = zeros16
        d_zero = pltpu.async_copy(zb, pool.at[pl.ds(s * RPW, RPW)], sW)
        d_ids.wait()
        d_seq.wait()
        d_len.wait()
        lv = lenv[...]
        gds = [pltpu.async_copy(tab.at[idq5.at[f]], rows5.at[f], sG)
               for f, tab in enumerate([ut, gt, at_, ot, zt])]
        ga = [None] * RPW
        for r in range(RPW):
            ga[r] = pltpu.async_copy(it.at[sidx.at[r, pl.ds(0, CH)]],
                                     buf.at[r, pl.ds(0, CH)], gsems[2 * r])

            @pl.when(lv[r] > CH)
            def _(r=r):
                pltpu.async_copy(it.at[sidx.at[r, pl.ds(CH, CHB)]],
                                 buf.at[r, pl.ds(CH, CHB)], gsems[2 * r + 1])
        for f in range(5):
            gds[f].wait()
        wds = [pltpu.async_copy(rows5.at[f], feats.at[f, pl.ds(base, RPW)], sW)
               for f in range(5)]
        dumpv = jnp.full((16,), DUMP, jnp.int32)
        d_zero.wait()
        sca = [None] * RPW
        for r in range(RPW):
            lb = jnp.broadcast_to(lv[r], (16,))
            slotv = jnp.full((16,), s * RPW + r, jnp.int32)
            for rr in range(NCH):
                for ch in range(CH // 16):
                    jv = lax.iota(jnp.int32, 16) + (rr * CH + ch * 16)
                    idx2[r, rr, pl.ds(ch * 16, 16)] = jnp.where(
                        jv < lb, slotv, dumpv)
            ga[r].wait()
            sca[r] = pltpu.async_copy(buf.at[r, pl.ds(0, CH)],
                                      pool.at[idx2.at[r, 0]], sS, add=True)

            @pl.when(lv[r] > CH)
            def _(r=r):
                pltpu.make_async_copy(it.at[sidx.at[r, pl.ds(CH, CHB)]],
                                      buf.at[r, pl.ds(CH, CHB)],
                                      gsems[2 * r + 1]).wait()
                pltpu.async_copy(buf.at[r, pl.ds(CH, CH)],
                                 pool.at[idx2.at[r, 1]], sS, add=True)
        for r in range(RPW):
            sca[r].wait()

            @pl.when(lv[r] > CH)
            def _(r=r):
                pltpu.make_async_copy(buf.at[r, pl.ds(CH, CH)],
                                      pool.at[idx2.at[r, 1]], sS).wait()
        pltpu.sync_copy(pool.at[pl.ds(s * RPW, RPW)], pb)
        pltpu.sync_copy(pb, feats.at[5, pl.ds(base, RPW)])
        for d in wds:
            d.wait()


_sc1 = pl.kernel(
    _sc1_body,
    out_type=jax.ShapeDtypeStruct((6, B, D), jnp.float32),
    mesh=_mesh,
    compiler_params=_params,
    scratch_types=[
        pltpu.VMEM((5, RPW), jnp.int32),
        pltpu.VMEM((5, RPW, D), jnp.float32),
        pltpu.VMEM((RPW, L), jnp.int32),
        pltpu.VMEM((16,), jnp.int32),
        pltpu.VMEM((RPW, SEQ_PAD, D), jnp.float32),
        pltpu.VMEM((RPW, NCH, CH), jnp.int32),
        pltpu.VMEM((RPW, D), jnp.float32),
        pltpu.VMEM((RPW, D), jnp.float32),
        pltpu.VMEM_SHARED((16 * RPW + 8, D), jnp.float32),
        pltpu.SemaphoreType.DMA,
        pltpu.SemaphoreType.DMA,
        pltpu.SemaphoreType.DMA,
        pltpu.SemaphoreType.DMA,
    ] + [pltpu.SemaphoreType.DMA] * (2 * RPW),
)


def _tc_body(x_ref, w_ref, b_ref, itT_ref, out_ref):
    u = jnp.dot(x_ref[0], w_ref[0], preferred_element_type=jnp.float32,
                precision=lax.Precision.HIGHEST)
    for f in range(1, 6):
        u = u + jnp.dot(x_ref[f], w_ref[f], preferred_element_type=jnp.float32,
                        precision=lax.Precision.HIGHEST)
    u = jnp.maximum(u + b_ref[...], 0.0)
    out_ref[...] = jnp.dot(u, itT_ref[...], preferred_element_type=jnp.float32,
                           precision=lax.Precision.HIGHEST)


def _sc2_body(sflat, pn, out, pnb, srow8, ob, sA, sG):
    c = lax.axis_index("c")
    s = lax.axis_index("s")
    wid = s * 2 + c

    @pl.when(wid < NWORK)
    def _():
        base = wid * RPW
        pltpu.async_copy(pn.at[pl.ds(base, RPW)], pnb, sA).wait()
        nchunks = NPN // 16
        for r in range(RPW):
            off = (base + r) * V_PAD
            for ch in range(nchunks):
                pnb[r, pl.ds(ch * 16, 16)] = pnb[r, pl.ds(ch * 16, 16)] + off
        gds = [pltpu.async_copy(sflat.at[pnb.at[r]], srow8.at[r], sG)
               for r in range(RPW)]
        for d in gds:
            d.wait()
        lanemask = (lax.iota(jnp.int32, 16) + (nchunks - 1) * 16) < (N + 1)
        for r in range(RPW):
            chunks = [srow8[r, pl.ds(ch * 16, 16)] for ch in range(nchunks)]
            neg_inf = jnp.full((16,), -3e38, jnp.float32)
            chunks[-1] = jnp.where(lanemask, chunks[-1], neg_inf)
            m = chunks[0]
            for ch in range(1, nchunks):
                m = jnp.maximum(m, chunks[ch])
            ms = jnp.max(m)
            es = [jnp.exp(cv - ms) for cv in chunks]
            es[-1] = jnp.where(lanemask, es[-1], jnp.zeros((16,), jnp.float32))
            tot = es[0]
            for ch in range(1, nchunks):
                tot = tot + es[ch]
            denom = jnp.broadcast_to(jnp.sum(tot), (16,))
            inv = jnp.ones((16,), jnp.float32) / denom
            for ch in range(nchunks):
                ob[r, pl.ds(ch * 16, 16)] = es[ch] * inv
        pltpu.sync_copy(ob, out.at[pl.ds(base, RPW)])


_sc2 = pl.kernel(
    _sc2_body,
    out_type=jax.ShapeDtypeStruct((B, NPN), jnp.float32),
    mesh=_mesh,
    compiler_params=_params,
    scratch_types=[
        pltpu.VMEM((RPW, NPN), jnp.int32),
        pltpu.VMEM((RPW, NPN), jnp.float32),
        pltpu.VMEM((RPW, NPN), jnp.float32),
        pltpu.SemaphoreType.DMA,
        pltpu.SemaphoreType.DMA,
    ],
)


def kernel(user_id, gender, age, occupation, zip_code, user_click_item_seq,
           user_click_item_seq_length, pos_item_sample, neg_item_sample,
           user_emb, gender_emb, age_emb, occupation_emb, zip_emb, item_emb,
           W1, b1):
    i32 = lambda x: x.astype(jnp.int32)
    ids5 = jnp.stack([i32(user_id), i32(gender), i32(age), i32(occupation),
                      i32(zip_code)], axis=0)
    feats = _sc1(ids5, i32(user_click_item_seq),
                 i32(user_click_item_seq_length),
                 user_emb, gender_emb, age_emb, occupation_emb, zip_emb,
                 item_emb)
    itT = jnp.pad(item_emb, ((0, V_PAD - V_ITEM), (0, 0))).T
    w6 = W1.reshape(6, D, D)
    scores = pl.pallas_call(
        _tc_body,
        out_shape=jax.ShapeDtypeStruct((B, V_PAD), jnp.float32),
    )(feats, w6, b1.reshape(1, D), itT)
    pn = jnp.concatenate([i32(pos_item_sample), i32(neg_item_sample)], axis=1)
    pn = jnp.pad(pn, ((0, 0), (0, NPN - (N + 1))))
    probs = _sc2(scores.reshape(B * V_PAD), pn)
    return probs[:, :N + 1].reshape(B, 1, N + 1)
